# baseline (speedup 1.0000x reference)
import functools

import jax
import jax.numpy as jnp
from jax import lax
from jax.experimental import pallas as pl
from jax.experimental.pallas import tpu as pltpu
from jax.experimental.pallas import tpu_sc as plsc


def _fold_table(table, W, bias):
    def body(t_ref, w_ref, b_ref, o_ref):
        o_ref[...] = (
            lax.dot_general(
                t_ref[...], w_ref[...],
                dimension_numbers=(((1,), (1,)), ((), ())),
                preferred_element_type=jnp.float32,
            )
            + b_ref[...]
        )

    return pl.pallas_call(
        body,
        out_shape=jax.ShapeDtypeStruct((table.shape[0], W.shape[0]), jnp.float32),
    )(table, W, bias.reshape(1, -1))


def _sc_gather_tiled(t_flat, idx_w, bsz, seqlen, h):
    info = plsc.get_sparse_core_info()
    nw = info.num_cores * info.num_subcores
    bpw = bsz // nw
    btw_n = bpw // 128
    cl = 2
    chunks = seqlen // cl
    groups = cl * btw_n * 8

    mesh = plsc.VectorSubcoreMesh(core_axis_name="c", subcore_axis_name="s")

    @functools.partial(
        pl.kernel,
        mesh=mesh,
        compiler_params=pltpu.CompilerParams(
            use_tc_tiling_on_sc=False, needs_layout_passes=False
        ),
        out_type=jax.ShapeDtypeStruct(
            (seqlen, h // 8, bsz // 128, 8, 128), jnp.float32
        ),
        scratch_types=[
            pltpu.VMEM((t_flat.shape[0],), jnp.float32),
            pltpu.VMEM((2, cl, bpw), jnp.int32),
            pltpu.VMEM((2, cl, h // 8, btw_n, 8, 128), jnp.float32),
            pltpu.SemaphoreType.DMA,
            pltpu.SemaphoreType.DMA,
        ],
    )
    def gather_kernel(t_hbm, idx_hbm, out_hbm, t_v, idx_v, tile_v, sem_in, sem_out):
        wid = lax.axis_index("s") * info.num_cores + lax.axis_index("c")
        btw = wid * btw_n
        pltpu.sync_copy(t_hbm, t_v)
        pltpu.async_copy(
            idx_hbm.at[wid, pl.ds(0, cl)], idx_v.at[0], sem_in
        ).wait()

        def out_slice(c):
            return out_hbm.at[pl.ds(c * cl, cl), :, pl.ds(btw, btw_n)]

        def do_chunk(c, p):
            @pl.when(c + 1 < chunks)
            def _():
                pltpu.async_copy(
                    idx_hbm.at[wid, pl.ds((c + 1) * cl, cl)],
                    idx_v.at[1 - p],
                    sem_in,
                )

            @pl.when(c >= 2)
            def _():
                pltpu.make_async_copy(
                    tile_v.at[p], out_slice(c - 2), sem_out
                ).wait()

            tv = tile_v.at[p]
            ixp = idx_v.at[p]

            def group_body(g, carry2):
                ll = lax.shift_right_logical(g, 5)
                btl = lax.shift_right_logical(g, 3) & 3
                j16 = (g & 31) * 16
                idx16 = ixp[ll, pl.ds(j16, 16)]
                vbase = idx16 * h
                blg = (g & 7) * 16
                for hh in range(h):
                    val = plsc.load_gather(t_v, [vbase + hh])
                    tv[ll, hh // 8, btl, hh % 8, pl.ds(blg, 16)] = val
                return carry2

            lax.fori_loop(0, groups, group_body, 0)

            pltpu.async_copy(tv, out_slice(c), sem_out)

        def two_chunks(cc, carry):
            c = cc * 2

            @pl.when(c > 0)
            def _():
                pltpu.make_async_copy(
                    idx_hbm.at[wid, pl.ds(c * cl, cl)], idx_v.at[0], sem_in
                ).wait()

            do_chunk(c, 0)

            pltpu.make_async_copy(
                idx_hbm.at[wid, pl.ds((c + 1) * cl, cl)], idx_v.at[1], sem_in
            ).wait()
            do_chunk(c + 1, 1)
            return carry

        lax.fori_loop(0, chunks // 2, two_chunks, 0)

        for c, p in ((chunks - 2, 0), (chunks - 1, 1)):
            pltpu.make_async_copy(tile_v.at[p], out_slice(c), sem_out).wait()

    return gather_kernel(t_flat, idx_w)


def kernel(x, embed_table, W, b):
    bsz, seqlen = x.shape
    h = embed_table.shape[1]
    t_folded = _fold_table(embed_table, W, b)
    nw = 32
    idx_w = (
        x.astype(jnp.int32).T.reshape(seqlen, nw, bsz // nw).swapaxes(0, 1)
    )
    out5 = _sc_gather_tiled(t_folded.reshape(-1), idx_w, bsz, seqlen, h)
    return out5.transpose(2, 4, 0, 1, 3).reshape(bsz, seqlen, h)


# speedup vs baseline: 1.7089x; 1.7089x over previous
import functools

import jax
import jax.numpy as jnp
from jax import lax
from jax.experimental import pallas as pl
from jax.experimental.pallas import tpu as pltpu
from jax.experimental.pallas import tpu_sc as plsc


def _fold_table(table, W, bias):
    def body(t_ref, w_ref, b_ref, o_ref):
        o_ref[...] = (
            lax.dot_general(
                t_ref[...], w_ref[...],
                dimension_numbers=(((1,), (1,)), ((), ())),
                preferred_element_type=jnp.float32,
            )
            + b_ref[...]
        )

    return pl.pallas_call(
        body,
        out_shape=jax.ShapeDtypeStruct((table.shape[0], W.shape[0]), jnp.float32),
    )(table, W, bias.reshape(1, -1))


def _sc_gather_tiled(t_flat, idx_w, bsz, seqlen, h):
    info = plsc.get_sparse_core_info()
    nw = info.num_cores * info.num_subcores
    bpw = bsz // nw
    btw_n = bpw // 128
    cl = 2
    chunks = seqlen // cl
    groups = cl * btw_n * 8

    mesh = plsc.VectorSubcoreMesh(core_axis_name="c", subcore_axis_name="s")

    @functools.partial(
        pl.kernel,
        mesh=mesh,
        compiler_params=pltpu.CompilerParams(
            use_tc_tiling_on_sc=False, needs_layout_passes=False
        ),
        out_type=jax.ShapeDtypeStruct(
            (seqlen, h // 8, bsz // 128, 8, 128), jnp.float32
        ),
        scratch_types=[
            pltpu.VMEM((t_flat.shape[0],), jnp.float32),
            pltpu.VMEM((2, cl, bpw), jnp.int32),
            pltpu.VMEM((2, cl, h // 8, btw_n, 8, 128), jnp.float32),
            pltpu.SemaphoreType.DMA,
            pltpu.SemaphoreType.DMA,
        ],
    )
    def gather_kernel(t_hbm, idx_hbm, out_hbm, t_v, idx_v, tile_v, sem_in, sem_out):
        wid = lax.axis_index("s") * info.num_cores + lax.axis_index("c")
        btw = wid * btw_n
        pltpu.sync_copy(t_hbm, t_v)
        pltpu.async_copy(
            idx_hbm.at[wid, pl.ds(0, cl)], idx_v.at[0], sem_in
        ).wait()

        def out_slice(c):
            return out_hbm.at[pl.ds(c * cl, cl), :, pl.ds(btw, btw_n)]

        def do_chunk(c, p):
            @pl.when(c + 1 < chunks)
            def _():
                pltpu.async_copy(
                    idx_hbm.at[wid, pl.ds((c + 1) * cl, cl)],
                    idx_v.at[1 - p],
                    sem_in,
                )

            @pl.when(c >= 2)
            def _():
                pltpu.make_async_copy(
                    tile_v.at[p], out_slice(c - 2), sem_out
                ).wait()

            tv = tile_v.at[p]
            ixp = idx_v.at[p]

            def group_body(g, carry2):
                ll = lax.shift_right_logical(g, 5)
                btl = lax.shift_right_logical(g, 3) & 3
                j16 = (g & 31) * 16
                idx16 = ixp[ll, pl.ds(j16, 16)]
                vbase = idx16 * h
                blg = (g & 7) * 16
                vals = [plsc.load_gather(t_v, [vbase + hh]) for hh in range(h)]
                for hh in range(h):
                    tv[ll, hh // 8, btl, hh % 8, pl.ds(blg, 16)] = vals[hh]
                return carry2

            lax.fori_loop(0, groups, group_body, 0)

            pltpu.async_copy(tv, out_slice(c), sem_out)

        def two_chunks(cc, carry):
            c = cc * 2

            @pl.when(c > 0)
            def _():
                pltpu.make_async_copy(
                    idx_hbm.at[wid, pl.ds(c * cl, cl)], idx_v.at[0], sem_in
                ).wait()

            do_chunk(c, 0)

            pltpu.make_async_copy(
                idx_hbm.at[wid, pl.ds((c + 1) * cl, cl)], idx_v.at[1], sem_in
            ).wait()
            do_chunk(c + 1, 1)
            return carry

        lax.fori_loop(0, chunks // 2, two_chunks, 0)

        for c, p in ((chunks - 2, 0), (chunks - 1, 1)):
            pltpu.make_async_copy(tile_v.at[p], out_slice(c), sem_out).wait()

    return gather_kernel(t_flat, idx_w)


def kernel(x, embed_table, W, b):
    bsz, seqlen = x.shape
    h = embed_table.shape[1]
    t_folded = _fold_table(embed_table, W, b)
    nw = 32
    idx_w = (
        x.astype(jnp.int32).T.reshape(seqlen, nw, bsz // nw).swapaxes(0, 1)
    )
    out5 = _sc_gather_tiled(t_folded.reshape(-1), idx_w, bsz, seqlen, h)
    return out5.transpose(2, 4, 0, 1, 3).reshape(bsz, seqlen, h)


# speedup vs baseline: 8.0222x; 4.6945x over previous
import functools

import jax
import jax.numpy as jnp
from jax import lax
from jax.experimental import pallas as pl
from jax.experimental.pallas import tpu as pltpu
from jax.experimental.pallas import tpu_sc as plsc


def _fold_table(table, W, bias):
    def body(t_ref, w_ref, b_ref, o_ref):
        o_ref[...] = (
            lax.dot_general(
                w_ref[...], t_ref[...],
                dimension_numbers=(((1,), (1,)), ((), ())),
                preferred_element_type=jnp.float32,
            )
            + b_ref[...]
        )

    return pl.pallas_call(
        body,
        out_shape=jax.ShapeDtypeStruct((W.shape[0], table.shape[0]), jnp.float32),
    )(table, W, bias.reshape(-1, 1))


def _sc_gather_tiled(t_flat, idx_w, bsz, seqlen, h):
    voc = t_flat.shape[0] // h
    info = plsc.get_sparse_core_info()
    nw = info.num_cores * info.num_subcores
    bpw = bsz // nw
    btw_n = bpw // 128
    cl = 2
    chunks = seqlen // cl
    groups = cl * btw_n * 8

    mesh = plsc.VectorSubcoreMesh(core_axis_name="c", subcore_axis_name="s")

    @functools.partial(
        pl.kernel,
        mesh=mesh,
        compiler_params=pltpu.CompilerParams(
            use_tc_tiling_on_sc=False, needs_layout_passes=False
        ),
        out_type=jax.ShapeDtypeStruct(
            (seqlen, h // 8, bsz // 128, 8, 128), jnp.float32
        ),
        scratch_types=[
            pltpu.VMEM((t_flat.shape[0],), jnp.float32),
            pltpu.VMEM((2, cl, bpw), jnp.int32),
            pltpu.VMEM((2, cl, h // 8, btw_n, 8, 128), jnp.float32),
            pltpu.SemaphoreType.DMA,
            pltpu.SemaphoreType.DMA,
        ],
    )
    def gather_kernel(t_hbm, idx_hbm, out_hbm, t_v, idx_v, tile_v, sem_in, sem_out):
        wid = lax.axis_index("s") * info.num_cores + lax.axis_index("c")
        btw = wid * btw_n
        pltpu.sync_copy(t_hbm, t_v)
        pltpu.async_copy(
            idx_hbm.at[wid, pl.ds(0, cl)], idx_v.at[0], sem_in
        ).wait()

        def out_slice(c):
            return out_hbm.at[pl.ds(c * cl, cl), :, pl.ds(btw, btw_n)]

        def do_chunk(c, p):
            @pl.when(c + 1 < chunks)
            def _():
                pltpu.async_copy(
                    idx_hbm.at[wid, pl.ds((c + 1) * cl, cl)],
                    idx_v.at[1 - p],
                    sem_in,
                )

            @pl.when(c >= 2)
            def _():
                pltpu.make_async_copy(
                    tile_v.at[p], out_slice(c - 2), sem_out
                ).wait()

            tv = tile_v.at[p]
            ixp = idx_v.at[p]

            def group_body(g, carry2):
                ll = lax.shift_right_logical(g, 5)
                btl = lax.shift_right_logical(g, 3) & 3
                j16 = (g & 31) * 16
                idx16 = ixp[ll, pl.ds(j16, 16)]
                blg = (g & 7) * 16
                vals = [
                    plsc.load_gather(t_v, [idx16 + hh * voc]) for hh in range(h)
                ]
                for hh in range(h):
                    tv[ll, hh // 8, btl, hh % 8, pl.ds(blg, 16)] = vals[hh]
                return carry2

            lax.fori_loop(0, groups, group_body, 0)

            pltpu.async_copy(tv, out_slice(c), sem_out)

        def two_chunks(cc, carry):
            c = cc * 2

            @pl.when(c > 0)
            def _():
                pltpu.make_async_copy(
                    idx_hbm.at[wid, pl.ds(c * cl, cl)], idx_v.at[0], sem_in
                ).wait()

            do_chunk(c, 0)

            pltpu.make_async_copy(
                idx_hbm.at[wid, pl.ds((c + 1) * cl, cl)], idx_v.at[1], sem_in
            ).wait()
            do_chunk(c + 1, 1)
            return carry

        lax.fori_loop(0, chunks // 2, two_chunks, 0)

        for c, p in ((chunks - 2, 0), (chunks - 1, 1)):
            pltpu.make_async_copy(tile_v.at[p], out_slice(c), sem_out).wait()

    return gather_kernel(t_flat, idx_w)


def kernel(x, embed_table, W, b):
    bsz, seqlen = x.shape
    h = embed_table.shape[1]
    t_folded = _fold_table(embed_table, W, b)
    nw = 32
    idx_w = (
        x.astype(jnp.int32).T.reshape(seqlen, nw, bsz // nw).swapaxes(0, 1)
    )
    out5 = _sc_gather_tiled(t_folded.reshape(-1), idx_w, bsz, seqlen, h)
    return out5.transpose(2, 4, 0, 1, 3).reshape(bsz, seqlen, h)


# speedup vs baseline: 11.8582x; 1.4782x over previous
import functools

import jax
import jax.numpy as jnp
from jax import lax
from jax.experimental import pallas as pl
from jax.experimental.pallas import tpu as pltpu
from jax.experimental.pallas import tpu_sc as plsc


def _fold_table(table, W, bias):
    def body(t_ref, w_ref, b_ref, o_ref):
        o_ref[...] = (
            lax.dot_general(
                w_ref[...], t_ref[...],
                dimension_numbers=(((1,), (1,)), ((), ())),
                preferred_element_type=jnp.float32,
            )
            + b_ref[...]
        )

    return pl.pallas_call(
        body,
        out_shape=jax.ShapeDtypeStruct((W.shape[0], table.shape[0]), jnp.float32),
    )(table, W, bias.reshape(-1, 1))


def _sc_gather_tiled(t_flat, idx_w, bsz, seqlen, h):
    voc = t_flat.shape[0] // h
    info = plsc.get_sparse_core_info()
    nw = info.num_cores * info.num_subcores
    bpw = bsz // nw
    btw_n = bpw // 128
    cl = 2
    chunks = seqlen // cl
    groups = cl * btw_n * 8

    mesh = plsc.VectorSubcoreMesh(core_axis_name="c", subcore_axis_name="s")

    @functools.partial(
        pl.kernel,
        mesh=mesh,
        compiler_params=pltpu.CompilerParams(
            use_tc_tiling_on_sc=False, needs_layout_passes=False
        ),
        out_type=jax.ShapeDtypeStruct(
            (seqlen, h // 8, bsz // 128, 8, 128), jnp.float32
        ),
        scratch_types=[
            pltpu.VMEM((t_flat.shape[0],), jnp.float32),
            pltpu.VMEM((2, cl, bpw), jnp.int32),
            pltpu.VMEM((2, cl, h // 8, btw_n, 8, 128), jnp.float32),
            pltpu.SemaphoreType.DMA,
            pltpu.SemaphoreType.DMA,
        ],
    )
    def gather_kernel(t_hbm, idx_hbm, out_hbm, t_v, idx_v, tile_v, sem_in, sem_out):
        wid = lax.axis_index("s") * info.num_cores + lax.axis_index("c")
        btw = wid * btw_n
        pltpu.sync_copy(t_hbm, t_v)
        pltpu.async_copy(
            idx_hbm.at[wid, pl.ds(0, cl)], idx_v.at[0], sem_in
        ).wait()

        def out_slice(c):
            return out_hbm.at[pl.ds(c * cl, cl), :, pl.ds(btw, btw_n)]

        def do_chunk(c, p):
            @pl.when(c + 1 < chunks)
            def _():
                pltpu.async_copy(
                    idx_hbm.at[wid, pl.ds((c + 1) * cl, cl)],
                    idx_v.at[1 - p],
                    sem_in,
                )

            @pl.when(c >= 2)
            def _():
                pltpu.make_async_copy(
                    tile_v.at[p], out_slice(c - 2), sem_out
                ).wait()

            tv = tile_v.at[p]
            ixp = idx_v.at[p]
            hf = h // 2

            def load16(g):
                ll = lax.shift_right_logical(g, 5)
                j16 = (g & 31) * 16
                return ixp[ll, pl.ds(j16, 16)]

            def store(g, hh, val):
                ll = lax.shift_right_logical(g, 5)
                btl = lax.shift_right_logical(g, 3) & 3
                blg = (g & 7) * 16
                tv[ll, hh // 8, btl, hh % 8, pl.ds(blg, 16)] = val

            def group_body(g, carry2):
                idx16 = load16(g)
                cur = []
                for i in range(hf):
                    cur.append(plsc.load_gather(t_v, [idx16 + i * voc]))
                    store(g - 1, hf + i, carry2[i])
                cur2 = []
                for i in range(hf):
                    cur2.append(plsc.load_gather(t_v, [idx16 + (hf + i) * voc]))
                    store(g, i, cur[i])
                return tuple(cur2)

            idx0 = load16(0)
            cur = [plsc.load_gather(t_v, [idx0 + i * voc]) for i in range(hf)]
            cur2 = []
            for i in range(hf):
                cur2.append(plsc.load_gather(t_v, [idx0 + (hf + i) * voc]))
                store(0, i, cur[i])
            last = lax.fori_loop(1, groups, group_body, tuple(cur2))
            for i in range(hf):
                store(groups - 1, hf + i, last[i])

            pltpu.async_copy(tv, out_slice(c), sem_out)

        def two_chunks(cc, carry):
            c = cc * 2

            @pl.when(c > 0)
            def _():
                pltpu.make_async_copy(
                    idx_hbm.at[wid, pl.ds(c * cl, cl)], idx_v.at[0], sem_in
                ).wait()

            do_chunk(c, 0)

            pltpu.make_async_copy(
                idx_hbm.at[wid, pl.ds((c + 1) * cl, cl)], idx_v.at[1], sem_in
            ).wait()
            do_chunk(c + 1, 1)
            return carry

        lax.fori_loop(0, chunks // 2, two_chunks, 0)

        for c, p in ((chunks - 2, 0), (chunks - 1, 1)):
            pltpu.make_async_copy(tile_v.at[p], out_slice(c), sem_out).wait()

    return gather_kernel(t_flat, idx_w)


def kernel(x, embed_table, W, b):
    bsz, seqlen = x.shape
    h = embed_table.shape[1]
    t_folded = _fold_table(embed_table, W, b)
    nw = 32
    idx_w = (
        x.astype(jnp.int32).T.reshape(seqlen, nw, bsz // nw).swapaxes(0, 1)
    )
    out5 = _sc_gather_tiled(t_folded.reshape(-1), idx_w, bsz, seqlen, h)
    return out5.transpose(2, 4, 0, 1, 3).reshape(bsz, seqlen, h)


# speedup vs baseline: 11.8998x; 1.0035x over previous
import functools

import jax
import jax.numpy as jnp
from jax import lax
from jax.experimental import pallas as pl
from jax.experimental.pallas import tpu as pltpu
from jax.experimental.pallas import tpu_sc as plsc


def _fold_table(table, W, bias):
    def body(t_ref, w_ref, b_ref, o_ref):
        o_ref[...] = (
            lax.dot_general(
                w_ref[...], t_ref[...],
                dimension_numbers=(((1,), (1,)), ((), ())),
                preferred_element_type=jnp.float32,
            )
            + b_ref[...]
        )

    return pl.pallas_call(
        body,
        out_shape=jax.ShapeDtypeStruct((W.shape[0], table.shape[0]), jnp.float32),
    )(table, W, bias.reshape(-1, 1))


def _sc_gather_tiled(t_flat, idx_w, bsz, seqlen, h):
    voc = t_flat.shape[0] // h
    info = plsc.get_sparse_core_info()
    nw = info.num_cores * info.num_subcores
    bpw = bsz // nw
    btw_n = bpw // 128
    cl = 2
    chunks = seqlen // cl
    groups = cl * btw_n * 8

    mesh = plsc.VectorSubcoreMesh(core_axis_name="c", subcore_axis_name="s")

    @functools.partial(
        pl.kernel,
        mesh=mesh,
        compiler_params=pltpu.CompilerParams(
            use_tc_tiling_on_sc=False, needs_layout_passes=False
        ),
        out_type=jax.ShapeDtypeStruct(
            (seqlen, h // 8, bsz // 128, 8, 128), jnp.float32
        ),
        scratch_types=[
            pltpu.VMEM((t_flat.shape[0],), jnp.float32),
            pltpu.VMEM((2, cl * bpw), jnp.int32),
            pltpu.VMEM((2, cl, h // 8, btw_n, 8, 128), jnp.float32),
            pltpu.SemaphoreType.DMA,
            pltpu.SemaphoreType.DMA,
        ],
    )
    def gather_kernel(t_hbm, idx_hbm, out_hbm, t_v, idx_v, tile_v, sem_in, sem_out):
        wid = lax.axis_index("s") * info.num_cores + lax.axis_index("c")
        btw = wid * btw_n
        i0w = wid * seqlen * bpw
        pltpu.sync_copy(t_hbm, t_v)
        pltpu.async_copy(
            idx_hbm.at[pl.ds(i0w, cl * bpw)], idx_v.at[0], sem_in
        ).wait()

        def out_slice(c):
            return out_hbm.at[pl.ds(c * cl, cl), :, pl.ds(btw, btw_n)]

        def do_chunk(c, p):
            @pl.when(c + 1 < chunks)
            def _():
                pltpu.async_copy(
                    idx_hbm.at[pl.ds(i0w + (c + 1) * cl * bpw, cl * bpw)],
                    idx_v.at[1 - p],
                    sem_in,
                )

            @pl.when(c >= 2)
            def _():
                pltpu.make_async_copy(
                    tile_v.at[p], out_slice(c - 2), sem_out
                ).wait()

            tv = tile_v.at[p]
            ixp = idx_v.at[p]
            hf = h // 2

            def load16(g):
                return ixp[pl.ds(g * 16, 16)]

            def store(g, hh, val):
                ll = lax.shift_right_logical(g, 5)
                btl = lax.shift_right_logical(g, 3) & 3
                blg = (g & 7) * 16
                tv[ll, hh // 8, btl, hh % 8, pl.ds(blg, 16)] = val

            def group_body(g, carry2):
                idx16 = load16(g)
                cur = []
                for i in range(hf):
                    cur.append(plsc.load_gather(t_v, [idx16 + i * voc]))
                    store(g - 1, hf + i, carry2[i])
                cur2 = []
                for i in range(hf):
                    cur2.append(plsc.load_gather(t_v, [idx16 + (hf + i) * voc]))
                    store(g, i, cur[i])
                return tuple(cur2)

            idx0 = load16(0)
            cur = [plsc.load_gather(t_v, [idx0 + i * voc]) for i in range(hf)]
            cur2 = []
            for i in range(hf):
                cur2.append(plsc.load_gather(t_v, [idx0 + (hf + i) * voc]))
                store(0, i, cur[i])
            last = lax.fori_loop(1, groups, group_body, tuple(cur2))
            for i in range(hf):
                store(groups - 1, hf + i, last[i])

            pltpu.async_copy(tv, out_slice(c), sem_out)

        def two_chunks(cc, carry):
            c = cc * 2

            @pl.when(c > 0)
            def _():
                pltpu.make_async_copy(
                    idx_hbm.at[pl.ds(i0w + c * cl * bpw, cl * bpw)],
                    idx_v.at[0],
                    sem_in,
                ).wait()

            do_chunk(c, 0)

            pltpu.make_async_copy(
                idx_hbm.at[pl.ds(i0w + (c + 1) * cl * bpw, cl * bpw)],
                idx_v.at[1],
                sem_in,
            ).wait()
            do_chunk(c + 1, 1)
            return carry

        lax.fori_loop(0, chunks // 2, two_chunks, 0)

        for c, p in ((chunks - 2, 0), (chunks - 1, 1)):
            pltpu.make_async_copy(tile_v.at[p], out_slice(c), sem_out).wait()

    return gather_kernel(t_flat, idx_w)


def kernel(x, embed_table, W, b):
    bsz, seqlen = x.shape
    h = embed_table.shape[1]
    t_folded = _fold_table(embed_table, W, b)
    nw = 32
    idx_w = (
        x.astype(jnp.int32).T.reshape(seqlen, nw, bsz // nw)
        .swapaxes(0, 1).reshape(-1)
    )
    out5 = _sc_gather_tiled(t_folded.reshape(-1), idx_w, bsz, seqlen, h)
    return out5.transpose(2, 4, 0, 1, 3).reshape(bsz, seqlen, h)
